# baseline (speedup 1.0000x reference)
import functools

import jax
import jax.numpy as jnp
from jax import lax
from jax.experimental import pallas as pl
from jax.experimental.pallas import tpu as pltpu
from jax.experimental.pallas import tpu_sc as plsc

_N = 4096
_D = 128
_F = 2
_K = 32
_ALPHA = 3.0
_R = 256


def _make_sc_gather(vocab1, vocab2):
    info = plsc.get_sparse_core_info()
    nw = info.num_cores * info.num_subcores
    b_per_w = _N // nw
    mesh = plsc.VectorSubcoreMesh(core_axis_name="c", subcore_axis_name="s")

    @functools.partial(
        pl.kernel, mesh=mesh,
        out_type=[
            jax.ShapeDtypeStruct((_N, _D), jnp.float32),
            jax.ShapeDtypeStruct((_N, _D), jnp.float32),
        ],
        scratch_types=[
            pltpu.VMEM((b_per_w,), jnp.int32),
            pltpu.VMEM((b_per_w, _D), jnp.float32),
            pltpu.VMEM((b_per_w, _D), jnp.float32),
            pltpu.SemaphoreType.DMA,
        ],
    )
    def gather_k(idx_hbm, emb1_hbm, emb2_hbm, out1_hbm, out2_hbm,
                 idx_v, rows1_v, rows2_v, sem):
        wid = lax.axis_index("s") * info.num_cores + lax.axis_index("c")
        base = wid * b_per_w
        pltpu.sync_copy(idx_hbm.at[pl.ds(base, b_per_w)], idx_v)
        c1 = pltpu.async_copy(emb1_hbm.at[idx_v], rows1_v, sem)
        c2 = pltpu.async_copy(emb2_hbm.at[idx_v], rows2_v, sem)
        c1.wait()
        c2.wait()
        pltpu.sync_copy(rows1_v, out1_hbm.at[pl.ds(base, b_per_w)])
        pltpu.sync_copy(rows2_v, out2_hbm.at[pl.ds(base, b_per_w)])

    return gather_k


def _prep_kernel(nv1_ref, nv2_ref, w1_ref, b1_ref, w2_ref, b2_ref,
                 v1_ref, v2_ref):
    nv1 = nv1_ref[...]
    nv2 = nv2_ref[...]
    u1 = jax.lax.dot_general(nv1, w1_ref[0], (((1,), (1,)), ((), ())),
                             preferred_element_type=jnp.float32) + b1_ref[0]
    v1_ref[0] = jnp.tanh(_ALPHA * u1)
    u2 = jax.lax.dot_general(nv2, w2_ref[0], (((1,), (1,)), ((), ())),
                             preferred_element_type=jnp.float32) + b2_ref[0]
    v2_ref[0] = jnp.tanh(_ALPHA * u2)


def _topk_mask(adj0, adj0t):
    r = adj0.shape[0]
    bits_t = jax.lax.bitcast_convert_type(adj0t, jnp.int32)
    lo = jnp.zeros((1, r), jnp.int32)
    hi = jnp.full((1, r), 1 << 30, jnp.int32)

    def vbody(_, carry):
        lo, hi = carry
        mid = lo + ((hi - lo + 1) >> 1)
        cnt = jnp.sum(jnp.where(bits_t >= mid, 1.0, 0.0), axis=0,
                      keepdims=True)
        ok = cnt >= float(_K)
        return jnp.where(ok, mid, lo), jnp.where(ok, hi, mid)

    lo, hi = jax.lax.fori_loop(0, 30, vbody, (lo, hi), unroll=True)
    vk_t = lo
    cgt_t = jnp.sum(jnp.where(bits_t > vk_t, 1.0, 0.0), axis=0,
                    keepdims=True)
    vk = vk_t.reshape(r, 1)
    t_need = float(_K) - cgt_t.reshape(r, 1)
    bits = jax.lax.bitcast_convert_type(adj0, jnp.int32)
    gt = bits > vk
    eq = bits == vk
    c = _N // 128
    eq3 = eq.astype(jnp.bfloat16).reshape(r, c, 128)
    li = jax.lax.broadcasted_iota(jnp.int32, (128, 128), 0)
    lj = jax.lax.broadcasted_iota(jnp.int32, (128, 128), 1)
    m_incl = (li <= lj).astype(jnp.bfloat16)
    ci = jax.lax.broadcasted_iota(jnp.int32, (c, c), 0)
    cj = jax.lax.broadcasted_iota(jnp.int32, (c, c), 1)
    m_excl = (ci < cj).astype(jnp.bfloat16)
    pref = jax.lax.dot_general(eq3, m_incl, (((2,), (0,)), ((), ())),
                               preferred_element_type=jnp.float32)
    tot = jnp.sum(eq3.astype(jnp.float32), axis=2)
    off = jax.lax.dot_general(tot.astype(jnp.bfloat16), m_excl,
                              (((1,), (0,)), ((), ())),
                              preferred_element_type=jnp.float32)
    run = (pref + off[:, :, None]).reshape(r, _N)
    return gt | (eq & (run <= t_need))


def _main_kernel(v1_ref, v2_ref, out0_ref, out1_ref):
    rb = pl.program_id(0)
    outs = (out0_ref, out1_ref)
    for i in range(_F):
        v1f = v1_ref[i]
        v2f = v2_ref[i]
        v1b = v1_ref[i, pl.ds(rb * _R, _R), :]
        v2b = v2_ref[i, pl.ds(rb * _R, _R), :]
        s1 = jax.lax.dot_general(v1b, v2f, (((1,), (1,)), ((), ())),
                                 preferred_element_type=jnp.float32)
        s2 = jax.lax.dot_general(v2b, v1f, (((1,), (1,)), ((), ())),
                                 preferred_element_type=jnp.float32)
        adj0 = jnp.maximum(jnp.tanh(_ALPHA * (s1 - s2)), 0.0)
        s1t = jax.lax.dot_general(v2f, v1b, (((1,), (1,)), ((), ())),
                                  preferred_element_type=jnp.float32)
        s2t = jax.lax.dot_general(v1f, v2b, (((1,), (1,)), ((), ())),
                                  preferred_element_type=jnp.float32)
        adj0t = jnp.maximum(jnp.tanh(_ALPHA * (s1t - s2t)), 0.0)
        mask = _topk_mask(adj0, adj0t)
        outs[i][...] = jnp.where(mask, adj0, 0.0)


def kernel(idx, emb1, emb2, W1, b1, W2, b2):
    nv1, nv2 = _make_sc_gather(emb1.shape[0], emb2.shape[0])(idx, emb1, emb2)
    b1r = b1.reshape(_F, 1, _D)
    b2r = b2.reshape(_F, 1, _D)
    v1, v2 = pl.pallas_call(
        _prep_kernel,
        grid=(_F,),
        in_specs=[
            pl.BlockSpec((_N, _D), lambda i: (0, 0)),
            pl.BlockSpec((_N, _D), lambda i: (0, 0)),
            pl.BlockSpec((1, _D, _D), lambda i: (i, 0, 0)),
            pl.BlockSpec((1, 1, _D), lambda i: (i, 0, 0)),
            pl.BlockSpec((1, _D, _D), lambda i: (i, 0, 0)),
            pl.BlockSpec((1, 1, _D), lambda i: (i, 0, 0)),
        ],
        out_specs=[
            pl.BlockSpec((1, _N, _D), lambda i: (i, 0, 0)),
            pl.BlockSpec((1, _N, _D), lambda i: (i, 0, 0)),
        ],
        out_shape=[
            jax.ShapeDtypeStruct((_F, _N, _D), jnp.float32),
            jax.ShapeDtypeStruct((_F, _N, _D), jnp.float32),
        ],
        compiler_params=pltpu.CompilerParams(
            dimension_semantics=("parallel",)),
    )(nv1, nv2, W1, b1r, W2, b2r)

    adj_out0, adj_out1 = pl.pallas_call(
        _main_kernel,
        grid=(_N // _R,),
        in_specs=[
            pl.BlockSpec((_F, _N, _D), lambda r: (0, 0, 0)),
            pl.BlockSpec((_F, _N, _D), lambda r: (0, 0, 0)),
        ],
        out_specs=[
            pl.BlockSpec((_R, _N), lambda r: (r, 0)),
            pl.BlockSpec((_R, _N), lambda r: (r, 0)),
        ],
        out_shape=[
            jax.ShapeDtypeStruct((_N, _N), jnp.float32),
            jax.ShapeDtypeStruct((_N, _N), jnp.float32),
        ],
        compiler_params=pltpu.CompilerParams(
            dimension_semantics=("parallel",)),
    )(v1, v2)
    return (adj_out0, adj_out1)


# speedup vs baseline: 1.4804x; 1.4804x over previous
import functools

import jax
import jax.numpy as jnp
from jax import lax
from jax.experimental import pallas as pl
from jax.experimental.pallas import tpu as pltpu
from jax.experimental.pallas import tpu_sc as plsc

_N = 4096
_D = 128
_F = 2
_K = 32
_ALPHA = 3.0
_R = 256


def _make_sc_gather(vocab1, vocab2):
    info = plsc.get_sparse_core_info()
    nw = info.num_cores * info.num_subcores
    b_per_w = _N // nw
    mesh = plsc.VectorSubcoreMesh(core_axis_name="c", subcore_axis_name="s")

    @functools.partial(
        pl.kernel, mesh=mesh,
        out_type=[
            jax.ShapeDtypeStruct((_N, _D), jnp.float32),
            jax.ShapeDtypeStruct((_N, _D), jnp.float32),
        ],
        scratch_types=[
            pltpu.VMEM((b_per_w,), jnp.int32),
            pltpu.VMEM((b_per_w, _D), jnp.float32),
            pltpu.VMEM((b_per_w, _D), jnp.float32),
            pltpu.SemaphoreType.DMA,
        ],
    )
    def gather_k(idx_hbm, emb1_hbm, emb2_hbm, out1_hbm, out2_hbm,
                 idx_v, rows1_v, rows2_v, sem):
        wid = lax.axis_index("s") * info.num_cores + lax.axis_index("c")
        base = wid * b_per_w
        pltpu.sync_copy(idx_hbm.at[pl.ds(base, b_per_w)], idx_v)
        c1 = pltpu.async_copy(emb1_hbm.at[idx_v], rows1_v, sem)
        c2 = pltpu.async_copy(emb2_hbm.at[idx_v], rows2_v, sem)
        c1.wait()
        c2.wait()
        pltpu.sync_copy(rows1_v, out1_hbm.at[pl.ds(base, b_per_w)])
        pltpu.sync_copy(rows2_v, out2_hbm.at[pl.ds(base, b_per_w)])

    return gather_k


def _prep_kernel(nv1_ref, nv2_ref, w1_ref, b1_ref, w2_ref, b2_ref,
                 v1_ref, v2_ref):
    nv1 = nv1_ref[...]
    nv2 = nv2_ref[...]
    u1 = jax.lax.dot_general(nv1, w1_ref[0], (((1,), (1,)), ((), ())),
                             preferred_element_type=jnp.float32) + b1_ref[0]
    v1_ref[0] = jnp.tanh(_ALPHA * u1)
    u2 = jax.lax.dot_general(nv2, w2_ref[0], (((1,), (1,)), ((), ())),
                             preferred_element_type=jnp.float32) + b2_ref[0]
    v2_ref[0] = jnp.tanh(_ALPHA * u2)


def _topk_mask(adj0):
    r = adj0.shape[0]
    bits = jax.lax.bitcast_convert_type(adj0, jnp.int32)
    lo = jnp.zeros((r, 1), jnp.int32)
    hi = jnp.full((r, 1), 1 << 30, jnp.int32)

    def vbody(_, carry):
        lo, hi = carry
        mid = lo + ((hi - lo + 1) >> 1)
        cnt = jnp.sum(jnp.where(bits >= mid, 1.0, 0.0), axis=1,
                      keepdims=True)
        ok = cnt >= float(_K)
        return jnp.where(ok, mid, lo), jnp.where(ok, hi, mid)

    lo, hi = jax.lax.fori_loop(0, 30, vbody, (lo, hi), unroll=True)
    vk = lo
    cgt = jnp.sum(jnp.where(bits > vk, 1.0, 0.0), axis=1, keepdims=True)
    t_need = float(_K) - cgt
    gt = bits > vk
    eq = bits == vk
    c = _N // 128
    eq3 = eq.astype(jnp.bfloat16).reshape(r, c, 128)
    li = jax.lax.broadcasted_iota(jnp.int32, (128, 128), 0)
    lj = jax.lax.broadcasted_iota(jnp.int32, (128, 128), 1)
    m_incl = (li <= lj).astype(jnp.bfloat16)
    ci = jax.lax.broadcasted_iota(jnp.int32, (c, c), 0)
    cj = jax.lax.broadcasted_iota(jnp.int32, (c, c), 1)
    m_excl = (ci < cj).astype(jnp.bfloat16)
    pref = jax.lax.dot_general(eq3, m_incl, (((2,), (0,)), ((), ())),
                               preferred_element_type=jnp.float32)
    tot = jnp.sum(eq3.astype(jnp.float32), axis=2)
    off = jax.lax.dot_general(tot.astype(jnp.bfloat16), m_excl,
                              (((1,), (0,)), ((), ())),
                              preferred_element_type=jnp.float32)
    run = (pref + off[:, :, None]).reshape(r, _N)
    return gt | (eq & (run <= t_need))


def _main_kernel(v1_ref, v2_ref, out0_ref, out1_ref):
    rb = pl.program_id(0)
    outs = (out0_ref, out1_ref)
    for i in range(_F):
        v1f = v1_ref[i]
        v2f = v2_ref[i]
        v1b = v1_ref[i, pl.ds(rb * _R, _R), :]
        v2b = v2_ref[i, pl.ds(rb * _R, _R), :]
        s1 = jax.lax.dot_general(v1b, v2f, (((1,), (1,)), ((), ())),
                                 preferred_element_type=jnp.float32)
        s2 = jax.lax.dot_general(v2b, v1f, (((1,), (1,)), ((), ())),
                                 preferred_element_type=jnp.float32)
        adj0 = jnp.maximum(jnp.tanh(_ALPHA * (s1 - s2)), 0.0)
        mask = _topk_mask(adj0)
        outs[i][...] = jnp.where(mask, adj0, 0.0)


def kernel(idx, emb1, emb2, W1, b1, W2, b2):
    nv1, nv2 = _make_sc_gather(emb1.shape[0], emb2.shape[0])(idx, emb1, emb2)
    b1r = b1.reshape(_F, 1, _D)
    b2r = b2.reshape(_F, 1, _D)
    v1, v2 = pl.pallas_call(
        _prep_kernel,
        grid=(_F,),
        in_specs=[
            pl.BlockSpec((_N, _D), lambda i: (0, 0)),
            pl.BlockSpec((_N, _D), lambda i: (0, 0)),
            pl.BlockSpec((1, _D, _D), lambda i: (i, 0, 0)),
            pl.BlockSpec((1, 1, _D), lambda i: (i, 0, 0)),
            pl.BlockSpec((1, _D, _D), lambda i: (i, 0, 0)),
            pl.BlockSpec((1, 1, _D), lambda i: (i, 0, 0)),
        ],
        out_specs=[
            pl.BlockSpec((1, _N, _D), lambda i: (i, 0, 0)),
            pl.BlockSpec((1, _N, _D), lambda i: (i, 0, 0)),
        ],
        out_shape=[
            jax.ShapeDtypeStruct((_F, _N, _D), jnp.float32),
            jax.ShapeDtypeStruct((_F, _N, _D), jnp.float32),
        ],
        compiler_params=pltpu.CompilerParams(
            dimension_semantics=("parallel",)),
    )(nv1, nv2, W1, b1r, W2, b2r)

    adj_out0, adj_out1 = pl.pallas_call(
        _main_kernel,
        grid=(_N // _R,),
        in_specs=[
            pl.BlockSpec((_F, _N, _D), lambda r: (0, 0, 0)),
            pl.BlockSpec((_F, _N, _D), lambda r: (0, 0, 0)),
        ],
        out_specs=[
            pl.BlockSpec((_R, _N), lambda r: (r, 0)),
            pl.BlockSpec((_R, _N), lambda r: (r, 0)),
        ],
        out_shape=[
            jax.ShapeDtypeStruct((_N, _N), jnp.float32),
            jax.ShapeDtypeStruct((_N, _N), jnp.float32),
        ],
        compiler_params=pltpu.CompilerParams(
            dimension_semantics=("parallel",)),
    )(v1, v2)
    return (adj_out0, adj_out1)


# speedup vs baseline: 1.6403x; 1.1080x over previous
import functools

import jax
import jax.numpy as jnp
from jax import lax
from jax.experimental import pallas as pl
from jax.experimental.pallas import tpu as pltpu
from jax.experimental.pallas import tpu_sc as plsc

_N = 4096
_D = 128
_F = 2
_K = 32
_ALPHA = 3.0
_R = 512


def _make_sc_gather(vocab1, vocab2):
    info = plsc.get_sparse_core_info()
    nw = info.num_cores * info.num_subcores
    b_per_w = _N // nw
    mesh = plsc.VectorSubcoreMesh(core_axis_name="c", subcore_axis_name="s")

    @functools.partial(
        pl.kernel, mesh=mesh,
        out_type=[
            jax.ShapeDtypeStruct((_N, _D), jnp.float32),
            jax.ShapeDtypeStruct((_N, _D), jnp.float32),
        ],
        scratch_types=[
            pltpu.VMEM((b_per_w,), jnp.int32),
            pltpu.VMEM((b_per_w, _D), jnp.float32),
            pltpu.VMEM((b_per_w, _D), jnp.float32),
            pltpu.SemaphoreType.DMA,
        ],
    )
    def gather_k(idx_hbm, emb1_hbm, emb2_hbm, out1_hbm, out2_hbm,
                 idx_v, rows1_v, rows2_v, sem):
        wid = lax.axis_index("s") * info.num_cores + lax.axis_index("c")
        base = wid * b_per_w
        pltpu.sync_copy(idx_hbm.at[pl.ds(base, b_per_w)], idx_v)
        c1 = pltpu.async_copy(emb1_hbm.at[idx_v], rows1_v, sem)
        c2 = pltpu.async_copy(emb2_hbm.at[idx_v], rows2_v, sem)
        c1.wait()
        c2.wait()
        pltpu.sync_copy(rows1_v, out1_hbm.at[pl.ds(base, b_per_w)])
        pltpu.sync_copy(rows2_v, out2_hbm.at[pl.ds(base, b_per_w)])

    return gather_k


def _prep_kernel(nv1_ref, nv2_ref, w1_ref, b1_ref, w2_ref, b2_ref,
                 v1_ref, v2_ref):
    nv1 = nv1_ref[...]
    nv2 = nv2_ref[...]
    u1 = jax.lax.dot_general(nv1, w1_ref[0], (((1,), (1,)), ((), ())),
                             preferred_element_type=jnp.float32) + b1_ref[0]
    v1_ref[0] = jnp.tanh(_ALPHA * u1)
    u2 = jax.lax.dot_general(nv2, w2_ref[0], (((1,), (1,)), ((), ())),
                             preferred_element_type=jnp.float32) + b2_ref[0]
    v2_ref[0] = jnp.tanh(_ALPHA * u2)


def _topk_mask(adj0):
    r = adj0.shape[0]
    bits = jax.lax.bitcast_convert_type(adj0, jnp.int32)
    lo = jnp.zeros((r, 1), jnp.int32)
    hi = jnp.full((r, 1), 1 << 30, jnp.int32)

    def vbody(_, carry):
        lo, hi = carry
        mid = lo + ((hi - lo + 1) >> 1)
        cnt = jnp.sum(jnp.where(bits >= mid, 1.0, 0.0), axis=1,
                      keepdims=True)
        ok = cnt >= float(_K)
        return jnp.where(ok, mid, lo), jnp.where(ok, hi, mid)

    lo, hi = jax.lax.fori_loop(0, 30, vbody, (lo, hi), unroll=True)
    vk = lo
    cgt = jnp.sum(jnp.where(bits > vk, 1.0, 0.0), axis=1, keepdims=True)
    t_need = float(_K) - cgt
    gt = bits > vk
    eq = bits == vk
    c = _N // 128
    eq3 = eq.astype(jnp.bfloat16).reshape(r, c, 128)
    li = jax.lax.broadcasted_iota(jnp.int32, (128, 128), 0)
    lj = jax.lax.broadcasted_iota(jnp.int32, (128, 128), 1)
    m_incl = (li <= lj).astype(jnp.bfloat16)
    ci = jax.lax.broadcasted_iota(jnp.int32, (c, c), 0)
    cj = jax.lax.broadcasted_iota(jnp.int32, (c, c), 1)
    m_excl = (ci < cj).astype(jnp.bfloat16)
    pref = jax.lax.dot_general(eq3, m_incl, (((2,), (0,)), ((), ())),
                               preferred_element_type=jnp.float32)
    tot = jnp.sum(eq3.astype(jnp.float32), axis=2)
    off = jax.lax.dot_general(tot.astype(jnp.bfloat16), m_excl,
                              (((1,), (0,)), ((), ())),
                              preferred_element_type=jnp.float32)
    run = (pref + off[:, :, None]).reshape(r, _N)
    return gt | (eq & (run <= t_need))


def _main_kernel(v1_ref, v2_ref, out0_ref, out1_ref):
    rb = pl.program_id(0)
    outs = (out0_ref, out1_ref)
    for i in range(_F):
        v1f = v1_ref[i]
        v2f = v2_ref[i]
        v1b = v1_ref[i, pl.ds(rb * _R, _R), :]
        v2b = v2_ref[i, pl.ds(rb * _R, _R), :]
        s1 = jax.lax.dot_general(v1b, v2f, (((1,), (1,)), ((), ())),
                                 preferred_element_type=jnp.float32)
        s2 = jax.lax.dot_general(v2b, v1f, (((1,), (1,)), ((), ())),
                                 preferred_element_type=jnp.float32)
        adj0 = jnp.maximum(jnp.tanh(_ALPHA * (s1 - s2)), 0.0)
        mask = _topk_mask(adj0)
        outs[i][...] = jnp.where(mask, adj0, 0.0)


def kernel(idx, emb1, emb2, W1, b1, W2, b2):
    nv1, nv2 = _make_sc_gather(emb1.shape[0], emb2.shape[0])(idx, emb1, emb2)
    b1r = b1.reshape(_F, 1, _D)
    b2r = b2.reshape(_F, 1, _D)
    v1, v2 = pl.pallas_call(
        _prep_kernel,
        grid=(_F,),
        in_specs=[
            pl.BlockSpec((_N, _D), lambda i: (0, 0)),
            pl.BlockSpec((_N, _D), lambda i: (0, 0)),
            pl.BlockSpec((1, _D, _D), lambda i: (i, 0, 0)),
            pl.BlockSpec((1, 1, _D), lambda i: (i, 0, 0)),
            pl.BlockSpec((1, _D, _D), lambda i: (i, 0, 0)),
            pl.BlockSpec((1, 1, _D), lambda i: (i, 0, 0)),
        ],
        out_specs=[
            pl.BlockSpec((1, _N, _D), lambda i: (i, 0, 0)),
            pl.BlockSpec((1, _N, _D), lambda i: (i, 0, 0)),
        ],
        out_shape=[
            jax.ShapeDtypeStruct((_F, _N, _D), jnp.float32),
            jax.ShapeDtypeStruct((_F, _N, _D), jnp.float32),
        ],
        compiler_params=pltpu.CompilerParams(
            dimension_semantics=("parallel",)),
    )(nv1, nv2, W1, b1r, W2, b2r)

    adj_out0, adj_out1 = pl.pallas_call(
        _main_kernel,
        grid=(_N // _R,),
        in_specs=[
            pl.BlockSpec((_F, _N, _D), lambda r: (0, 0, 0)),
            pl.BlockSpec((_F, _N, _D), lambda r: (0, 0, 0)),
        ],
        out_specs=[
            pl.BlockSpec((_R, _N), lambda r: (r, 0)),
            pl.BlockSpec((_R, _N), lambda r: (r, 0)),
        ],
        out_shape=[
            jax.ShapeDtypeStruct((_N, _N), jnp.float32),
            jax.ShapeDtypeStruct((_N, _N), jnp.float32),
        ],
        compiler_params=pltpu.CompilerParams(
            dimension_semantics=("parallel",)),
    )(v1, v2)
    return (adj_out0, adj_out1)
